# baseline (speedup 1.0000x reference)
import functools

import jax
import jax.numpy as jnp
from jax import lax
from jax.experimental import pallas as pl
from jax.experimental.pallas import tpu as pltpu
from jax.experimental.pallas import tpu_sc as plsc

N = 10000
D = 128
H = 128
C = 2
G = 128

NC = 2
NS = 16
NW = NC * NS
K = 128
NPAD = 10240
RPT = NPAD // NS
BLK = 1024
NBLK = NPAD // BLK

_sc_mesh = plsc.VectorSubcoreMesh(core_axis_name="c", subcore_axis_name="s")
_sc_mesh1 = plsc.VectorSubcoreMesh(core_axis_name="c", subcore_axis_name="s",
                                   num_cores=1)


def _num_chunks(e_pad):
    return e_pad // (NW * K)


def _deg_body(nchunks, dst_hbm, ones_hbm, zeros_hbm, out_hbm,
              dst_v, ones_v, deg_sp, sem):
    c = lax.axis_index("c")
    s = lax.axis_index("s")
    w = s * NC + c
    pltpu.sync_copy(dst_hbm.at[pl.ds(w * nchunks, nchunks)], dst_v)
    pltpu.sync_copy(ones_hbm, ones_v)
    pltpu.sync_copy(zeros_hbm.at[pl.ds(s * RPT, RPT)],
                    deg_sp.at[pl.ds(s * RPT, RPT)])
    plsc.subcore_barrier()

    def body(j, carry):
        pltpu.sync_copy(ones_v, deg_sp.at[dst_v.at[j]], add=True)
        return carry

    lax.fori_loop(0, nchunks, body, 0)
    plsc.subcore_barrier()
    pltpu.sync_copy(deg_sp.at[pl.ds(s * RPT, RPT)],
                    out_hbm.at[c, pl.ds(s * RPT, RPT)])


def _make_deg_kernel(nchunks):
    return functools.partial(
        pl.kernel,
        out_type=jax.ShapeDtypeStruct((NC, NPAD, 128), jnp.float32),
        mesh=_sc_mesh,
        scratch_types=[
            pltpu.VMEM((nchunks, K), jnp.int32),
            pltpu.VMEM((K, 128), jnp.float32),
            pltpu.VMEM_SHARED((NPAD, 128), jnp.float32),
            pltpu.SemaphoreType.DMA,
        ],
    )(functools.partial(_deg_body, nchunks))


def _agg_body(n, total_rows, hs_hbm, sd_hbm, zeros_hbm, out_hbm,
              sd_r, buf_a, buf_b, agg_sp, gsem_a, gsem_b, isem):
    s = lax.axis_index("s")
    base = s * n
    trips = n
    npairs = trips // 2

    pltpu.sync_copy(zeros_hbm.at[pl.ds(s * RPT, RPT)],
                    agg_sp.at[pl.ds(s * RPT, RPT)])
    plsc.subcore_barrier()

    pltpu.sync_copy(sd_hbm.at[pl.ds(base, 2)], sd_r.at[pl.ds(0, 2)])
    pltpu.async_copy(hs_hbm.at[sd_r.at[0, 0]], buf_a, gsem_a)
    pltpu.async_copy(hs_hbm.at[sd_r.at[1, 0]], buf_b, gsem_b)
    pltpu.async_copy(sd_hbm.at[pl.ds(base + 2, 2)], sd_r.at[pl.ds(2, 2)], isem)

    def body(i, carry):
        j = base + 2 * i
        cur = (i % 2) * 2
        nxt = ((i + 1) % 2) * 2
        pltpu.make_async_copy(sd_hbm.at[pl.ds(base, 2)],
                              sd_r.at[pl.ds(0, 2)], isem).wait()

        pltpu.make_async_copy(hs_hbm.at[sd_r.at[cur, 0]], buf_a, gsem_a).wait()
        pltpu.sync_copy(buf_a, agg_sp.at[sd_r.at[cur, 1]], add=True)

        @pl.when(2 * i + 2 < trips)
        def _():
            pltpu.async_copy(hs_hbm.at[sd_r.at[nxt, 0]], buf_a, gsem_a)

        pltpu.make_async_copy(hs_hbm.at[sd_r.at[cur + 1, 0]], buf_b,
                              gsem_b).wait()
        pltpu.sync_copy(buf_b, agg_sp.at[sd_r.at[cur + 1, 1]], add=True)

        @pl.when(2 * i + 3 < trips)
        def _():
            pltpu.async_copy(hs_hbm.at[sd_r.at[nxt + 1, 0]], buf_b, gsem_b)

        @pl.when(i + 1 < npairs)
        def _():
            row = jnp.minimum(j + 4, total_rows - 2)
            pltpu.async_copy(sd_hbm.at[pl.ds(row, 2)],
                             sd_r.at[pl.ds(cur, 2)], isem)
        return carry

    lax.fori_loop(0, npairs, body, 0)
    plsc.subcore_barrier()
    pltpu.sync_copy(agg_sp.at[pl.ds(s * RPT, RPT)],
                    out_hbm.at[pl.ds(s * RPT, RPT)])


def _make_agg_kernel(n, total_rows):
    return functools.partial(
        pl.kernel,
        out_type=jax.ShapeDtypeStruct((NPAD, 128), jnp.float32),
        mesh=_sc_mesh1,
        scratch_types=[
            pltpu.VMEM((4, 2, K), jnp.int32),
            pltpu.VMEM((K, 128), jnp.float32),
            pltpu.VMEM((K, 128), jnp.float32),
            pltpu.VMEM_SHARED((NPAD, 128), jnp.float32),
            pltpu.SemaphoreType.DMA,
            pltpu.SemaphoreType.DMA,
            pltpu.SemaphoreType.DMA,
        ],
    )(functools.partial(_agg_body, n, total_rows))


def _tca_body(x_ref, deg_ref, w1_ref, hs_ref, dinv_ref):
    deg = deg_ref[0, :, 0:1] + deg_ref[1, :, 0:1] + 1.0
    dinv = lax.rsqrt(deg)
    h = jnp.dot(x_ref[...], w1_ref[...], preferred_element_type=jnp.float32)
    hs_ref[...] = h * dinv
    dinv_ref[...] = dinv


def _tca(x_pad, deg2, w1):
    return pl.pallas_call(
        _tca_body,
        grid=(NBLK,),
        in_specs=[
            pl.BlockSpec((BLK, D), lambda i: (i, 0)),
            pl.BlockSpec((NC, BLK, 128), lambda i: (0, i, 0)),
            pl.BlockSpec((D, H), lambda i: (0, 0)),
        ],
        out_specs=[
            pl.BlockSpec((BLK, H), lambda i: (i, 0)),
            pl.BlockSpec((BLK, 1), lambda i: (i, 0)),
        ],
        out_shape=[
            jax.ShapeDtypeStruct((NPAD, H), jnp.float32),
            jax.ShapeDtypeStruct((NPAD, 1), jnp.float32),
        ],
    )(x_pad, deg2, w1)


def _tcb_body(agg_ref, hs1_ref, dinv_ref, b1_ref, w2_ref, hs2_ref):
    dinv = dinv_ref[...]
    tot = agg_ref[...] + hs1_ref[...]
    h1r = jnp.maximum(dinv * tot + b1_ref[...], 0.0)
    hs2_ref[...] = dinv * jnp.dot(h1r, w2_ref[...],
                                  preferred_element_type=jnp.float32)


def _tcb(agg1, hs1, dinv, b1, w2):
    return pl.pallas_call(
        _tcb_body,
        grid=(NBLK,),
        in_specs=[
            pl.BlockSpec((BLK, H), lambda i: (i, 0)),
            pl.BlockSpec((BLK, H), lambda i: (i, 0)),
            pl.BlockSpec((BLK, 1), lambda i: (i, 0)),
            pl.BlockSpec((1, H), lambda i: (0, 0)),
            pl.BlockSpec((H, H), lambda i: (0, 0)),
        ],
        out_specs=pl.BlockSpec((BLK, H), lambda i: (i, 0)),
        out_shape=jax.ShapeDtypeStruct((NPAD, H), jnp.float32),
    )(agg1, hs1, dinv, b1, w2)


def _tcc_body(agg_ref, hs2_ref, dinv_ref, b2_ref, batch_ref,
              wm1_ref, bm1_ref, wm2_ref, bm2_ref, out_ref,
              pool_ref, cnt_ref):
    i = pl.program_id(0)
    h2 = dinv_ref[...] * (agg_ref[...] + hs2_ref[...]) + b2_ref[...]
    bb = batch_ref[0]
    gid = lax.broadcasted_iota(jnp.int32, (G, BLK), 0)
    eq = (gid == bb).astype(jnp.float32)
    ps = jnp.dot(eq, h2, preferred_element_type=jnp.float32)
    cs = jnp.sum(eq, axis=1, keepdims=True)

    @pl.when(i == 0)
    def _():
        pool_ref[...] = ps
        cnt_ref[...] = jnp.broadcast_to(cs, (G, H))

    @pl.when(i > 0)
    def _():
        pool_ref[...] += ps
        cnt_ref[...] += jnp.broadcast_to(cs, (G, H))

    @pl.when(i == pl.num_programs(0) - 1)
    def _():
        pooled = pool_ref[...] / jnp.maximum(cnt_ref[...], 1.0)
        a = jnp.maximum(
            jnp.dot(pooled, wm1_ref[...], preferred_element_type=jnp.float32)
            + bm1_ref[...], 0.0)
        out_ref[...] = (jnp.dot(a, wm2_ref[...],
                                preferred_element_type=jnp.float32)
                        + bm2_ref[...])


def _tcc(agg2, hs2, dinv, b2, batch_r, wm1, bm1, wm2p, bm2p):
    return pl.pallas_call(
        _tcc_body,
        grid=(NBLK,),
        in_specs=[
            pl.BlockSpec((BLK, H), lambda i: (i, 0)),
            pl.BlockSpec((BLK, H), lambda i: (i, 0)),
            pl.BlockSpec((BLK, 1), lambda i: (i, 0)),
            pl.BlockSpec((1, H), lambda i: (0, 0)),
            pl.BlockSpec((1, 1, BLK), lambda i: (i, 0, 0)),
            pl.BlockSpec((H, H), lambda i: (0, 0)),
            pl.BlockSpec((1, H), lambda i: (0, 0)),
            pl.BlockSpec((H, H), lambda i: (0, 0)),
            pl.BlockSpec((1, H), lambda i: (0, 0)),
        ],
        out_specs=pl.BlockSpec((G, H), lambda i: (0, 0)),
        out_shape=jax.ShapeDtypeStruct((G, H), jnp.float32),
        scratch_shapes=[
            pltpu.VMEM((G, H), jnp.float32),
            pltpu.VMEM((G, H), jnp.float32),
        ],
    )(agg2, hs2, dinv, b2, batch_r, wm1, bm1, wm2p, bm2p)


def kernel(x, edge_index, batch, W1, b1, W2, b2, Wm1, bm1, Wm2, bm2):
    e = edge_index.shape[1]
    epw = -(-e // NW)
    nchunks = -(-epw // K)
    if nchunks % 2:
        nchunks += 1
    e_pad = NW * nchunks * K

    src = jnp.concatenate(
        [edge_index[0], jnp.full((e_pad - e,), N, dtype=jnp.int32)])
    dst = jnp.concatenate(
        [edge_index[1], jnp.full((e_pad - e,), N, dtype=jnp.int32)])
    src2 = src.reshape(NW * nchunks, K)
    dst2 = dst.reshape(NW * nchunks, K)
    sd3 = jnp.stack([src2, dst2], axis=1)

    n_per = 2 * nchunks

    x_pad = jnp.concatenate(
        [x, jnp.zeros((NPAD - N, D), dtype=jnp.float32)])
    batch_pad = jnp.concatenate(
        [batch, jnp.full((NPAD - N,), G, dtype=jnp.int32)]).reshape(NBLK, 1, BLK)

    ones128 = jnp.ones((K, 128), dtype=jnp.float32)
    zeros128 = jnp.zeros((NPAD, 128), dtype=jnp.float32)

    b1r = b1.reshape(1, H)
    b2r = b2.reshape(1, H)
    bm1r = bm1.reshape(1, H)
    wm2p = jnp.zeros((H, H), dtype=jnp.float32).at[:, :C].set(Wm2)
    bm2p = jnp.zeros((1, H), dtype=jnp.float32).at[0, :C].set(bm2)

    deg2 = _make_deg_kernel(nchunks)(dst2, ones128, zeros128)
    hs1, dinv = _tca(x_pad, deg2, W1)
    total_rows = NW * nchunks
    agg1 = _make_agg_kernel(n_per, total_rows)(hs1, sd3, zeros128)
    hs2 = _tcb(agg1, hs1, dinv, b1r, W2)
    agg2 = _make_agg_kernel(n_per, total_rows)(hs2, sd3, zeros128)
    outp = _tcc(agg2, hs2, dinv, b2r, batch_pad, Wm1, bm1r, wm2p, bm2p)
    return outp[:, :C]


# speedup vs baseline: 1.0151x; 1.0151x over previous
import functools

import jax
import jax.numpy as jnp
from jax import lax
from jax.experimental import pallas as pl
from jax.experimental.pallas import tpu as pltpu
from jax.experimental.pallas import tpu_sc as plsc

N = 10000
D = 128
H = 128
C = 2
G = 128

NC = 2
NS = 16
NW = NC * NS
K = 128
NPAD = 10240
RPT = NPAD // NS
BLK = 1024
NBLK = NPAD // BLK

_sc_mesh = plsc.VectorSubcoreMesh(core_axis_name="c", subcore_axis_name="s")
_sc_mesh1 = plsc.VectorSubcoreMesh(core_axis_name="c", subcore_axis_name="s",
                                   num_cores=1)


def _num_chunks(e_pad):
    return e_pad // (NW * K)


def _deg_body(nchunks, dst_hbm, ones_hbm, zeros_hbm, out_hbm,
              dst_v, ones_v, deg_sp, sem):
    c = lax.axis_index("c")
    s = lax.axis_index("s")
    w = s * NC + c
    pltpu.sync_copy(dst_hbm.at[pl.ds(w * nchunks, nchunks)], dst_v)
    pltpu.sync_copy(ones_hbm, ones_v)
    pltpu.sync_copy(zeros_hbm.at[pl.ds(s * RPT, RPT)],
                    deg_sp.at[pl.ds(s * RPT, RPT)])
    plsc.subcore_barrier()

    def body(j, carry):
        pltpu.sync_copy(ones_v, deg_sp.at[dst_v.at[j]], add=True)
        return carry

    lax.fori_loop(0, nchunks, body, 0)
    plsc.subcore_barrier()
    pltpu.sync_copy(deg_sp.at[pl.ds(s * RPT, RPT)],
                    out_hbm.at[c, pl.ds(s * RPT, RPT)])


def _make_deg_kernel(nchunks):
    return functools.partial(
        pl.kernel,
        out_type=jax.ShapeDtypeStruct((NC, NPAD, 128), jnp.float32),
        mesh=_sc_mesh,
        scratch_types=[
            pltpu.VMEM((nchunks, K), jnp.int32),
            pltpu.VMEM((K, 128), jnp.float32),
            pltpu.VMEM_SHARED((NPAD, 128), jnp.float32),
            pltpu.SemaphoreType.DMA,
        ],
    )(functools.partial(_deg_body, nchunks))


def _agg_body(n, total_rows, hs_hbm, sd_hbm, zeros_hbm, out_hbm,
              sd_r, buf_a, buf_b, agg_sp, gsem_a, gsem_b, isem):
    c = lax.axis_index("c")
    s = lax.axis_index("s")
    base = s * n
    trips = jnp.where(c == 0, n, 0)
    npairs = trips // 2

    @pl.when(trips > 0)
    def _():
        pltpu.sync_copy(zeros_hbm.at[pl.ds(s * RPT, RPT)],
                        agg_sp.at[pl.ds(s * RPT, RPT)])
    plsc.subcore_barrier()

    @pl.when(trips > 0)
    def _():
        pltpu.sync_copy(sd_hbm.at[pl.ds(base, 2)], sd_r.at[pl.ds(0, 2)])
        pltpu.async_copy(hs_hbm.at[sd_r.at[0, 0]], buf_a, gsem_a)
        pltpu.async_copy(hs_hbm.at[sd_r.at[1, 0]], buf_b, gsem_b)
        pltpu.async_copy(sd_hbm.at[pl.ds(base + 2, 2)], sd_r.at[pl.ds(2, 2)],
                         isem)

    def body(i, carry):
        j = base + 2 * i
        cur = (i % 2) * 2
        nxt = ((i + 1) % 2) * 2
        pltpu.make_async_copy(sd_hbm.at[pl.ds(base, 2)],
                              sd_r.at[pl.ds(0, 2)], isem).wait()

        pltpu.make_async_copy(hs_hbm.at[sd_r.at[cur, 0]], buf_a, gsem_a).wait()
        pltpu.sync_copy(buf_a, agg_sp.at[sd_r.at[cur, 1]], add=True)

        @pl.when(2 * i + 2 < trips)
        def _():
            pltpu.async_copy(hs_hbm.at[sd_r.at[nxt, 0]], buf_a, gsem_a)

        pltpu.make_async_copy(hs_hbm.at[sd_r.at[cur + 1, 0]], buf_b,
                              gsem_b).wait()
        pltpu.sync_copy(buf_b, agg_sp.at[sd_r.at[cur + 1, 1]], add=True)

        @pl.when(2 * i + 3 < trips)
        def _():
            pltpu.async_copy(hs_hbm.at[sd_r.at[nxt + 1, 0]], buf_b, gsem_b)

        @pl.when(i + 1 < npairs)
        def _():
            row = jnp.minimum(j + 4, total_rows - 2)
            pltpu.async_copy(sd_hbm.at[pl.ds(row, 2)],
                             sd_r.at[pl.ds(cur, 2)], isem)
        return carry

    lax.fori_loop(0, npairs, body, 0)
    plsc.subcore_barrier()

    @pl.when(trips > 0)
    def _():
        pltpu.sync_copy(agg_sp.at[pl.ds(s * RPT, RPT)],
                        out_hbm.at[pl.ds(s * RPT, RPT)])


def _make_agg_kernel(n, total_rows):
    return functools.partial(
        pl.kernel,
        out_type=jax.ShapeDtypeStruct((NPAD, 128), jnp.float32),
        mesh=_sc_mesh,
        scratch_types=[
            pltpu.VMEM((4, 2, K), jnp.int32),
            pltpu.VMEM((K, 128), jnp.float32),
            pltpu.VMEM((K, 128), jnp.float32),
            pltpu.VMEM_SHARED((NPAD, 128), jnp.float32),
            pltpu.SemaphoreType.DMA,
            pltpu.SemaphoreType.DMA,
            pltpu.SemaphoreType.DMA,
        ],
    )(functools.partial(_agg_body, n, total_rows))


def _tca_body(x_ref, deg_ref, w1_ref, hs_ref, dinv_ref):
    deg = deg_ref[0, :, 0:1] + deg_ref[1, :, 0:1] + 1.0
    dinv = lax.rsqrt(deg)
    h = jnp.dot(x_ref[...], w1_ref[...], preferred_element_type=jnp.float32)
    hs_ref[...] = h * dinv
    dinv_ref[...] = dinv


def _tca(x_pad, deg2, w1):
    return pl.pallas_call(
        _tca_body,
        grid=(NBLK,),
        in_specs=[
            pl.BlockSpec((BLK, D), lambda i: (i, 0)),
            pl.BlockSpec((NC, BLK, 128), lambda i: (0, i, 0)),
            pl.BlockSpec((D, H), lambda i: (0, 0)),
        ],
        out_specs=[
            pl.BlockSpec((BLK, H), lambda i: (i, 0)),
            pl.BlockSpec((BLK, 1), lambda i: (i, 0)),
        ],
        out_shape=[
            jax.ShapeDtypeStruct((NPAD, H), jnp.float32),
            jax.ShapeDtypeStruct((NPAD, 1), jnp.float32),
        ],
    )(x_pad, deg2, w1)


def _tcb_body(agg_ref, hs1_ref, dinv_ref, b1_ref, w2_ref, hs2_ref):
    dinv = dinv_ref[...]
    tot = agg_ref[...] + hs1_ref[...]
    h1r = jnp.maximum(dinv * tot + b1_ref[...], 0.0)
    hs2_ref[...] = dinv * jnp.dot(h1r, w2_ref[...],
                                  preferred_element_type=jnp.float32)


def _tcb(agg1, hs1, dinv, b1, w2):
    return pl.pallas_call(
        _tcb_body,
        grid=(NBLK,),
        in_specs=[
            pl.BlockSpec((BLK, H), lambda i: (i, 0)),
            pl.BlockSpec((BLK, H), lambda i: (i, 0)),
            pl.BlockSpec((BLK, 1), lambda i: (i, 0)),
            pl.BlockSpec((1, H), lambda i: (0, 0)),
            pl.BlockSpec((H, H), lambda i: (0, 0)),
        ],
        out_specs=pl.BlockSpec((BLK, H), lambda i: (i, 0)),
        out_shape=jax.ShapeDtypeStruct((NPAD, H), jnp.float32),
    )(agg1, hs1, dinv, b1, w2)


def _tcc_body(agg_ref, hs2_ref, dinv_ref, b2_ref, batch_ref,
              wm1_ref, bm1_ref, wm2_ref, bm2_ref, out_ref,
              pool_ref, cnt_ref):
    i = pl.program_id(0)
    h2 = dinv_ref[...] * (agg_ref[...] + hs2_ref[...]) + b2_ref[...]
    bb = batch_ref[0]
    gid = lax.broadcasted_iota(jnp.int32, (G, BLK), 0)
    eq = (gid == bb).astype(jnp.float32)
    ps = jnp.dot(eq, h2, preferred_element_type=jnp.float32)
    cs = jnp.sum(eq, axis=1, keepdims=True)

    @pl.when(i == 0)
    def _():
        pool_ref[...] = ps
        cnt_ref[...] = jnp.broadcast_to(cs, (G, H))

    @pl.when(i > 0)
    def _():
        pool_ref[...] += ps
        cnt_ref[...] += jnp.broadcast_to(cs, (G, H))

    @pl.when(i == pl.num_programs(0) - 1)
    def _():
        pooled = pool_ref[...] / jnp.maximum(cnt_ref[...], 1.0)
        a = jnp.maximum(
            jnp.dot(pooled, wm1_ref[...], preferred_element_type=jnp.float32)
            + bm1_ref[...], 0.0)
        out_ref[...] = (jnp.dot(a, wm2_ref[...],
                                preferred_element_type=jnp.float32)
                        + bm2_ref[...])


def _tcc(agg2, hs2, dinv, b2, batch_r, wm1, bm1, wm2p, bm2p):
    return pl.pallas_call(
        _tcc_body,
        grid=(NBLK,),
        in_specs=[
            pl.BlockSpec((BLK, H), lambda i: (i, 0)),
            pl.BlockSpec((BLK, H), lambda i: (i, 0)),
            pl.BlockSpec((BLK, 1), lambda i: (i, 0)),
            pl.BlockSpec((1, H), lambda i: (0, 0)),
            pl.BlockSpec((1, 1, BLK), lambda i: (i, 0, 0)),
            pl.BlockSpec((H, H), lambda i: (0, 0)),
            pl.BlockSpec((1, H), lambda i: (0, 0)),
            pl.BlockSpec((H, H), lambda i: (0, 0)),
            pl.BlockSpec((1, H), lambda i: (0, 0)),
        ],
        out_specs=pl.BlockSpec((G, H), lambda i: (0, 0)),
        out_shape=jax.ShapeDtypeStruct((G, H), jnp.float32),
        scratch_shapes=[
            pltpu.VMEM((G, H), jnp.float32),
            pltpu.VMEM((G, H), jnp.float32),
        ],
    )(agg2, hs2, dinv, b2, batch_r, wm1, bm1, wm2p, bm2p)


def kernel(x, edge_index, batch, W1, b1, W2, b2, Wm1, bm1, Wm2, bm2):
    e = edge_index.shape[1]
    epw = -(-e // NW)
    nchunks = -(-epw // K)
    if nchunks % 2:
        nchunks += 1
    e_pad = NW * nchunks * K

    src = jnp.concatenate(
        [edge_index[0], jnp.full((e_pad - e,), N, dtype=jnp.int32)])
    dst = jnp.concatenate(
        [edge_index[1], jnp.full((e_pad - e,), N, dtype=jnp.int32)])
    src2 = src.reshape(NW * nchunks, K)
    dst2 = dst.reshape(NW * nchunks, K)
    sd3 = jnp.stack([src2, dst2], axis=1)

    n_per = 2 * nchunks

    x_pad = jnp.concatenate(
        [x, jnp.zeros((NPAD - N, D), dtype=jnp.float32)])
    batch_pad = jnp.concatenate(
        [batch, jnp.full((NPAD - N,), G, dtype=jnp.int32)]).reshape(NBLK, 1, BLK)

    ones128 = jnp.ones((K, 128), dtype=jnp.float32)
    zeros128 = jnp.zeros((NPAD, 128), dtype=jnp.float32)

    b1r = b1.reshape(1, H)
    b2r = b2.reshape(1, H)
    bm1r = bm1.reshape(1, H)
    wm2p = jnp.zeros((H, H), dtype=jnp.float32).at[:, :C].set(Wm2)
    bm2p = jnp.zeros((1, H), dtype=jnp.float32).at[0, :C].set(bm2)

    deg2 = _make_deg_kernel(nchunks)(dst2, ones128, zeros128)
    hs1, dinv = _tca(x_pad, deg2, W1)
    total_rows = NW * nchunks
    agg1 = _make_agg_kernel(n_per, total_rows)(hs1, sd3, zeros128)
    hs2 = _tcb(agg1, hs1, dinv, b1r, W2)
    agg2 = _make_agg_kernel(n_per, total_rows)(hs2, sd3, zeros128)
    outp = _tcc(agg2, hs2, dinv, b2r, batch_pad, Wm1, bm1r, wm2p, bm2p)
    return outp[:, :C]


# speedup vs baseline: 1.1498x; 1.1326x over previous
import functools

import jax
import jax.numpy as jnp
from jax import lax
from jax.experimental import pallas as pl
from jax.experimental.pallas import tpu as pltpu
from jax.experimental.pallas import tpu_sc as plsc

N = 10000
D = 128
H = 128
C = 2
G = 128

NC = 2
NS = 16
NW = NC * NS
K = 128
NPAD = 10240
RPT = NPAD // NS
BLK = 1024
NBLK = NPAD // BLK

_sc_mesh = plsc.VectorSubcoreMesh(core_axis_name="c", subcore_axis_name="s")
_sc_mesh1 = plsc.VectorSubcoreMesh(core_axis_name="c", subcore_axis_name="s",
                                   num_cores=1)


def _num_chunks(e_pad):
    return e_pad // (NW * K)


def _deg_body(nchunks, dst_hbm, ones_hbm, zeros_hbm, out_hbm,
              dst_v, ones_v, deg_sp, sem):
    c = lax.axis_index("c")
    s = lax.axis_index("s")
    w = s * NC + c
    pltpu.sync_copy(dst_hbm.at[pl.ds(w * nchunks, nchunks)], dst_v)
    pltpu.sync_copy(ones_hbm, ones_v)
    pltpu.sync_copy(zeros_hbm.at[pl.ds(s * RPT, RPT)],
                    deg_sp.at[pl.ds(s * RPT, RPT)])
    plsc.subcore_barrier()

    def body(j, carry):
        pltpu.sync_copy(ones_v, deg_sp.at[dst_v.at[j]], add=True)
        return carry

    lax.fori_loop(0, nchunks, body, 0)
    plsc.subcore_barrier()
    pltpu.sync_copy(deg_sp.at[pl.ds(s * RPT, RPT)],
                    out_hbm.at[c, pl.ds(s * RPT, RPT)])


def _make_deg_kernel(nchunks):
    return functools.partial(
        pl.kernel,
        out_type=jax.ShapeDtypeStruct((NC, NPAD, 128), jnp.float32),
        mesh=_sc_mesh,
        scratch_types=[
            pltpu.VMEM((nchunks, K), jnp.int32),
            pltpu.VMEM((K, 128), jnp.float32),
            pltpu.VMEM_SHARED((NPAD, 128), jnp.float32),
            pltpu.SemaphoreType.DMA,
        ],
    )(functools.partial(_deg_body, nchunks))


def _agg_body(n0, n1, total_rows, hs_hbm, sd_hbm, out_hbm,
              sd_r, buf_a, buf_b, agg_sp, gsem_a, gsem_b, isem):
    c = lax.axis_index("c")
    s = lax.axis_index("s")
    base = jnp.where(c == 0, s * n0, NS * n0 + s * n1)
    trips = jnp.where(c == 0, n0, n1)
    npairs = trips // 2

    def zbody(r, carry):
        for col in range(8):
            buf_a[r, pl.ds(col * 16, 16)] = jnp.zeros((16,), jnp.float32)
        return carry

    lax.fori_loop(0, K, zbody, 0)
    for blk in range(RPT // K):
        pltpu.sync_copy(buf_a, agg_sp.at[pl.ds(s * RPT + blk * K, K)])
    plsc.subcore_barrier()

    @pl.when(trips > 0)
    def _():
        pltpu.sync_copy(sd_hbm.at[pl.ds(base, 2)], sd_r.at[pl.ds(0, 2)])
        pltpu.async_copy(hs_hbm.at[sd_r.at[0, 0]], buf_a, gsem_a)
        pltpu.async_copy(hs_hbm.at[sd_r.at[1, 0]], buf_b, gsem_b)
        pltpu.async_copy(sd_hbm.at[pl.ds(base + 2, 2)], sd_r.at[pl.ds(2, 2)],
                         isem)

    def body(i, carry):
        j = base + 2 * i
        cur = (i % 2) * 2
        nxt = ((i + 1) % 2) * 2
        pltpu.make_async_copy(sd_hbm.at[pl.ds(base, 2)],
                              sd_r.at[pl.ds(0, 2)], isem).wait()

        pltpu.make_async_copy(hs_hbm.at[sd_r.at[cur, 0]], buf_a, gsem_a).wait()
        pltpu.sync_copy(buf_a, agg_sp.at[sd_r.at[cur, 1]], add=True)

        @pl.when(2 * i + 2 < trips)
        def _():
            pltpu.async_copy(hs_hbm.at[sd_r.at[nxt, 0]], buf_a, gsem_a)

        pltpu.make_async_copy(hs_hbm.at[sd_r.at[cur + 1, 0]], buf_b,
                              gsem_b).wait()
        pltpu.sync_copy(buf_b, agg_sp.at[sd_r.at[cur + 1, 1]], add=True)

        @pl.when(2 * i + 3 < trips)
        def _():
            pltpu.async_copy(hs_hbm.at[sd_r.at[nxt + 1, 0]], buf_b, gsem_b)

        @pl.when(i + 1 < npairs)
        def _():
            row = jnp.minimum(j + 4, total_rows - 2)
            pltpu.async_copy(sd_hbm.at[pl.ds(row, 2)],
                             sd_r.at[pl.ds(cur, 2)], isem)
        return carry

    lax.fori_loop(0, npairs, body, 0)
    plsc.subcore_barrier()
    pltpu.sync_copy(agg_sp.at[pl.ds(s * RPT, RPT)],
                    out_hbm.at[c, pl.ds(s * RPT, RPT)])


def _make_agg_kernel(n0, n1, total_rows):
    return functools.partial(
        pl.kernel,
        out_type=jax.ShapeDtypeStruct((NC, NPAD, 128), jnp.float32),
        mesh=_sc_mesh,
        scratch_types=[
            pltpu.VMEM((4, 2, K), jnp.int32),
            pltpu.VMEM((K, 128), jnp.float32),
            pltpu.VMEM((K, 128), jnp.float32),
            pltpu.VMEM_SHARED((NPAD, 128), jnp.float32),
            pltpu.SemaphoreType.DMA,
            pltpu.SemaphoreType.DMA,
            pltpu.SemaphoreType.DMA,
        ],
    )(functools.partial(_agg_body, n0, n1, total_rows))


def _tca_body(x_ref, deg_ref, w1_ref, hs_ref, dinv_ref):
    deg = deg_ref[0, :, 0:1] + deg_ref[1, :, 0:1] + 1.0
    dinv = lax.rsqrt(deg)
    h = jnp.dot(x_ref[...], w1_ref[...], preferred_element_type=jnp.float32)
    hs_ref[...] = h * dinv
    dinv_ref[...] = dinv


def _tca(x_pad, deg2, w1):
    return pl.pallas_call(
        _tca_body,
        grid=(NBLK,),
        in_specs=[
            pl.BlockSpec((BLK, D), lambda i: (i, 0)),
            pl.BlockSpec((NC, BLK, 128), lambda i: (0, i, 0)),
            pl.BlockSpec((D, H), lambda i: (0, 0)),
        ],
        out_specs=[
            pl.BlockSpec((BLK, H), lambda i: (i, 0)),
            pl.BlockSpec((BLK, 1), lambda i: (i, 0)),
        ],
        out_shape=[
            jax.ShapeDtypeStruct((NPAD, H), jnp.float32),
            jax.ShapeDtypeStruct((NPAD, 1), jnp.float32),
        ],
    )(x_pad, deg2, w1)


def _tcb_body(agg_ref, hs1_ref, dinv_ref, b1_ref, w2_ref, hs2_ref):
    dinv = dinv_ref[...]
    tot = agg_ref[0] + agg_ref[1] + hs1_ref[...]
    h1r = jnp.maximum(dinv * tot + b1_ref[...], 0.0)
    hs2_ref[...] = dinv * jnp.dot(h1r, w2_ref[...],
                                  preferred_element_type=jnp.float32)


def _tcb(agg1, hs1, dinv, b1, w2):
    return pl.pallas_call(
        _tcb_body,
        grid=(NBLK,),
        in_specs=[
            pl.BlockSpec((NC, BLK, H), lambda i: (0, i, 0)),
            pl.BlockSpec((BLK, H), lambda i: (i, 0)),
            pl.BlockSpec((BLK, 1), lambda i: (i, 0)),
            pl.BlockSpec((1, H), lambda i: (0, 0)),
            pl.BlockSpec((H, H), lambda i: (0, 0)),
        ],
        out_specs=pl.BlockSpec((BLK, H), lambda i: (i, 0)),
        out_shape=jax.ShapeDtypeStruct((NPAD, H), jnp.float32),
    )(agg1, hs1, dinv, b1, w2)


def _tcc_body(agg_ref, hs2_ref, dinv_ref, b2_ref, batch_ref,
              wm1_ref, bm1_ref, wm2_ref, bm2_ref, out_ref,
              pool_ref, cnt_ref):
    i = pl.program_id(0)
    h2 = dinv_ref[...] * (agg_ref[0] + agg_ref[1] + hs2_ref[...]) + b2_ref[...]
    bb = batch_ref[0]
    gid = lax.broadcasted_iota(jnp.int32, (G, BLK), 0)
    eq = (gid == bb).astype(jnp.float32)
    ps = jnp.dot(eq, h2, preferred_element_type=jnp.float32)
    cs = jnp.sum(eq, axis=1, keepdims=True)

    @pl.when(i == 0)
    def _():
        pool_ref[...] = ps
        cnt_ref[...] = jnp.broadcast_to(cs, (G, H))

    @pl.when(i > 0)
    def _():
        pool_ref[...] += ps
        cnt_ref[...] += jnp.broadcast_to(cs, (G, H))

    @pl.when(i == pl.num_programs(0) - 1)
    def _():
        pooled = pool_ref[...] / jnp.maximum(cnt_ref[...], 1.0)
        a = jnp.maximum(
            jnp.dot(pooled, wm1_ref[...], preferred_element_type=jnp.float32)
            + bm1_ref[...], 0.0)
        out_ref[...] = (jnp.dot(a, wm2_ref[...],
                                preferred_element_type=jnp.float32)
                        + bm2_ref[...])


def _tcc(agg2, hs2, dinv, b2, batch_r, wm1, bm1, wm2p, bm2p):
    return pl.pallas_call(
        _tcc_body,
        grid=(NBLK,),
        in_specs=[
            pl.BlockSpec((NC, BLK, H), lambda i: (0, i, 0)),
            pl.BlockSpec((BLK, H), lambda i: (i, 0)),
            pl.BlockSpec((BLK, 1), lambda i: (i, 0)),
            pl.BlockSpec((1, H), lambda i: (0, 0)),
            pl.BlockSpec((1, 1, BLK), lambda i: (i, 0, 0)),
            pl.BlockSpec((H, H), lambda i: (0, 0)),
            pl.BlockSpec((1, H), lambda i: (0, 0)),
            pl.BlockSpec((H, H), lambda i: (0, 0)),
            pl.BlockSpec((1, H), lambda i: (0, 0)),
        ],
        out_specs=pl.BlockSpec((G, H), lambda i: (0, 0)),
        out_shape=jax.ShapeDtypeStruct((G, H), jnp.float32),
        scratch_shapes=[
            pltpu.VMEM((G, H), jnp.float32),
            pltpu.VMEM((G, H), jnp.float32),
        ],
    )(agg2, hs2, dinv, b2, batch_r, wm1, bm1, wm2p, bm2p)


def kernel(x, edge_index, batch, W1, b1, W2, b2, Wm1, bm1, Wm2, bm2):
    e = edge_index.shape[1]
    epw = -(-e // NW)
    nchunks = -(-epw // K)
    if nchunks % 2:
        nchunks += 1
    e_pad = NW * nchunks * K

    src = jnp.concatenate(
        [edge_index[0], jnp.full((e_pad - e,), N, dtype=jnp.int32)])
    dst = jnp.concatenate(
        [edge_index[1], jnp.full((e_pad - e,), N, dtype=jnp.int32)])
    src2 = src.reshape(NW * nchunks, K)
    dst2 = dst.reshape(NW * nchunks, K)
    sd3 = jnp.stack([src2, dst2], axis=1)

    per_pair = 2 * nchunks
    n0 = (per_pair * 13 // 20) // 2 * 2
    n1 = per_pair - n0

    x_pad = jnp.concatenate(
        [x, jnp.zeros((NPAD - N, D), dtype=jnp.float32)])
    batch_pad = jnp.concatenate(
        [batch, jnp.full((NPAD - N,), G, dtype=jnp.int32)]).reshape(NBLK, 1, BLK)

    ones128 = jnp.ones((K, 128), dtype=jnp.float32)
    zeros128 = jnp.zeros((NPAD, 128), dtype=jnp.float32)

    b1r = b1.reshape(1, H)
    b2r = b2.reshape(1, H)
    bm1r = bm1.reshape(1, H)
    wm2p = jnp.zeros((H, H), dtype=jnp.float32).at[:, :C].set(Wm2)
    bm2p = jnp.zeros((1, H), dtype=jnp.float32).at[0, :C].set(bm2)

    deg2 = _make_deg_kernel(nchunks)(dst2, ones128, zeros128)
    hs1, dinv = _tca(x_pad, deg2, W1)
    total_rows = NW * nchunks
    agg1 = _make_agg_kernel(n0, n1, total_rows)(hs1, sd3)
    hs2 = _tcb(agg1, hs1, dinv, b1r, W2)
    agg2 = _make_agg_kernel(n0, n1, total_rows)(hs2, sd3)
    outp = _tcc(agg2, hs2, dinv, b2r, batch_pad, Wm1, bm1r, wm2p, bm2p)
    return outp[:, :C]


# speedup vs baseline: 1.1547x; 1.0043x over previous
import functools

import jax
import jax.numpy as jnp
from jax import lax
from jax.experimental import pallas as pl
from jax.experimental.pallas import tpu as pltpu
from jax.experimental.pallas import tpu_sc as plsc

N = 10000
D = 128
H = 128
C = 2
G = 128

NC = 2
NS = 16
NW = NC * NS
K = 128
HK = 64
NPAD = 10240
RPT = NPAD // NS
BLK = 1024
NBLK = NPAD // BLK

_sc_mesh = plsc.VectorSubcoreMesh(core_axis_name="c", subcore_axis_name="s")
_sc_mesh1 = plsc.VectorSubcoreMesh(core_axis_name="c", subcore_axis_name="s",
                                   num_cores=1)


def _num_chunks(e_pad):
    return e_pad // (NW * K)


def _deg_body(nchunks, dst_hbm, ones_hbm, zeros_hbm, out_hbm,
              dst_v, ones_v, deg_sp, sem):
    c = lax.axis_index("c")
    s = lax.axis_index("s")
    w = s * NC + c
    pltpu.sync_copy(dst_hbm.at[pl.ds(w * nchunks, nchunks)], dst_v)
    pltpu.sync_copy(ones_hbm, ones_v)
    pltpu.sync_copy(zeros_hbm.at[pl.ds(s * RPT, RPT)],
                    deg_sp.at[pl.ds(s * RPT, RPT)])
    plsc.subcore_barrier()

    def body(j, carry):
        pltpu.sync_copy(ones_v, deg_sp.at[dst_v.at[j]], add=True)
        return carry

    lax.fori_loop(0, nchunks, body, 0)
    plsc.subcore_barrier()
    pltpu.sync_copy(deg_sp.at[pl.ds(s * RPT, RPT)],
                    out_hbm.at[c, pl.ds(s * RPT, RPT)])


def _make_deg_kernel(nchunks):
    return functools.partial(
        pl.kernel,
        out_type=jax.ShapeDtypeStruct((NC, NPAD, 128), jnp.float32),
        mesh=_sc_mesh,
        scratch_types=[
            pltpu.VMEM((nchunks, K), jnp.int32),
            pltpu.VMEM((K, 128), jnp.float32),
            pltpu.VMEM_SHARED((NPAD, 128), jnp.float32),
            pltpu.SemaphoreType.DMA,
        ],
    )(functools.partial(_deg_body, nchunks))


def _gather_halves(hs_hbm, sd_r, slot, buf, sem):
    buf_lo = buf.at[pl.ds(0, HK)]
    buf_hi = buf.at[pl.ds(HK, HK)]
    pltpu.async_copy(hs_hbm.at[sd_r.at[slot, 0, pl.ds(0, HK)]], buf_lo, sem)
    pltpu.async_copy(hs_hbm.at[sd_r.at[slot, 0, pl.ds(HK, HK)]], buf_hi, sem)


def _wait_halves(hs_hbm, sd_r, slot, buf, sem):
    buf_lo = buf.at[pl.ds(0, HK)]
    buf_hi = buf.at[pl.ds(HK, HK)]
    pltpu.make_async_copy(hs_hbm.at[sd_r.at[slot, 0, pl.ds(0, HK)]],
                          buf_lo, sem).wait()
    pltpu.make_async_copy(hs_hbm.at[sd_r.at[slot, 0, pl.ds(HK, HK)]],
                          buf_hi, sem).wait()


def _agg_body(n0, n1, total_rows, hs_hbm, sd_hbm, out_hbm,
              sd_r, buf_a, buf_b, agg_sp, gsem_a, gsem_b, isem):
    c = lax.axis_index("c")
    s = lax.axis_index("s")
    base = jnp.where(c == 0, s * n0, NS * n0 + s * n1)
    trips = jnp.where(c == 0, n0, n1)
    npairs = trips // 2

    def zbody(r, carry):
        for col in range(8):
            buf_a[r, pl.ds(col * 16, 16)] = jnp.zeros((16,), jnp.float32)
        return carry

    lax.fori_loop(0, K, zbody, 0)
    for blk in range(RPT // K):
        pltpu.sync_copy(buf_a, agg_sp.at[pl.ds(s * RPT + blk * K, K)])
    plsc.subcore_barrier()

    @pl.when(trips > 0)
    def _():
        pltpu.sync_copy(sd_hbm.at[pl.ds(base, 2)], sd_r.at[pl.ds(0, 2)])
        _gather_halves(hs_hbm, sd_r, 0, buf_a, gsem_a)
        _gather_halves(hs_hbm, sd_r, 1, buf_b, gsem_b)
        pltpu.async_copy(sd_hbm.at[pl.ds(base + 2, 2)], sd_r.at[pl.ds(2, 2)],
                         isem)

    def body(i, carry):
        j = base + 2 * i
        cur = (i % 2) * 2
        nxt = ((i + 1) % 2) * 2
        pltpu.make_async_copy(sd_hbm.at[pl.ds(base, 2)],
                              sd_r.at[pl.ds(0, 2)], isem).wait()

        _wait_halves(hs_hbm, sd_r, cur, buf_a, gsem_a)
        pltpu.sync_copy(buf_a, agg_sp.at[sd_r.at[cur, 1]], add=True)

        @pl.when(2 * i + 2 < trips)
        def _():
            _gather_halves(hs_hbm, sd_r, nxt, buf_a, gsem_a)

        _wait_halves(hs_hbm, sd_r, cur + 1, buf_b, gsem_b)
        pltpu.sync_copy(buf_b, agg_sp.at[sd_r.at[cur + 1, 1]], add=True)

        @pl.when(2 * i + 3 < trips)
        def _():
            _gather_halves(hs_hbm, sd_r, nxt + 1, buf_b, gsem_b)

        @pl.when(i + 1 < npairs)
        def _():
            row = jnp.minimum(j + 4, total_rows - 2)
            pltpu.async_copy(sd_hbm.at[pl.ds(row, 2)],
                             sd_r.at[pl.ds(cur, 2)], isem)
        return carry

    lax.fori_loop(0, npairs, body, 0)
    plsc.subcore_barrier()
    pltpu.sync_copy(agg_sp.at[pl.ds(s * RPT, RPT)],
                    out_hbm.at[c, pl.ds(s * RPT, RPT)])


def _make_agg_kernel(n0, n1, total_rows):
    return functools.partial(
        pl.kernel,
        out_type=jax.ShapeDtypeStruct((NC, NPAD, 128), jnp.float32),
        mesh=_sc_mesh,
        scratch_types=[
            pltpu.VMEM((4, 2, K), jnp.int32),
            pltpu.VMEM((K, 128), jnp.float32),
            pltpu.VMEM((K, 128), jnp.float32),
            pltpu.VMEM_SHARED((NPAD, 128), jnp.float32),
            pltpu.SemaphoreType.DMA,
            pltpu.SemaphoreType.DMA,
            pltpu.SemaphoreType.DMA,
        ],
    )(functools.partial(_agg_body, n0, n1, total_rows))


def _tca_body(x_ref, deg_ref, w1_ref, hs_ref, dinv_ref):
    deg = deg_ref[0, :, 0:1] + deg_ref[1, :, 0:1] + 1.0
    dinv = lax.rsqrt(deg)
    h = jnp.dot(x_ref[...], w1_ref[...], preferred_element_type=jnp.float32)
    hs_ref[...] = h * dinv
    dinv_ref[...] = dinv


def _tca(x_pad, deg2, w1):
    return pl.pallas_call(
        _tca_body,
        grid=(NBLK,),
        in_specs=[
            pl.BlockSpec((BLK, D), lambda i: (i, 0)),
            pl.BlockSpec((NC, BLK, 128), lambda i: (0, i, 0)),
            pl.BlockSpec((D, H), lambda i: (0, 0)),
        ],
        out_specs=[
            pl.BlockSpec((BLK, H), lambda i: (i, 0)),
            pl.BlockSpec((BLK, 1), lambda i: (i, 0)),
        ],
        out_shape=[
            jax.ShapeDtypeStruct((NPAD, H), jnp.float32),
            jax.ShapeDtypeStruct((NPAD, 1), jnp.float32),
        ],
    )(x_pad, deg2, w1)


def _tcb_body(agg_ref, hs1_ref, dinv_ref, b1_ref, w2_ref, hs2_ref):
    dinv = dinv_ref[...]
    tot = agg_ref[0] + agg_ref[1] + hs1_ref[...]
    h1r = jnp.maximum(dinv * tot + b1_ref[...], 0.0)
    hs2_ref[...] = dinv * jnp.dot(h1r, w2_ref[...],
                                  preferred_element_type=jnp.float32)


def _tcb(agg1, hs1, dinv, b1, w2):
    return pl.pallas_call(
        _tcb_body,
        grid=(NBLK,),
        in_specs=[
            pl.BlockSpec((NC, BLK, H), lambda i: (0, i, 0)),
            pl.BlockSpec((BLK, H), lambda i: (i, 0)),
            pl.BlockSpec((BLK, 1), lambda i: (i, 0)),
            pl.BlockSpec((1, H), lambda i: (0, 0)),
            pl.BlockSpec((H, H), lambda i: (0, 0)),
        ],
        out_specs=pl.BlockSpec((BLK, H), lambda i: (i, 0)),
        out_shape=jax.ShapeDtypeStruct((NPAD, H), jnp.float32),
    )(agg1, hs1, dinv, b1, w2)


def _tcc_body(agg_ref, hs2_ref, dinv_ref, b2_ref, batch_ref,
              wm1_ref, bm1_ref, wm2_ref, bm2_ref, out_ref,
              pool_ref, cnt_ref):
    i = pl.program_id(0)
    h2 = dinv_ref[...] * (agg_ref[0] + agg_ref[1] + hs2_ref[...]) + b2_ref[...]
    bb = batch_ref[0]
    gid = lax.broadcasted_iota(jnp.int32, (G, BLK), 0)
    eq = (gid == bb).astype(jnp.float32)
    ps = jnp.dot(eq, h2, preferred_element_type=jnp.float32)
    cs = jnp.sum(eq, axis=1, keepdims=True)

    @pl.when(i == 0)
    def _():
        pool_ref[...] = ps
        cnt_ref[...] = jnp.broadcast_to(cs, (G, H))

    @pl.when(i > 0)
    def _():
        pool_ref[...] += ps
        cnt_ref[...] += jnp.broadcast_to(cs, (G, H))

    @pl.when(i == pl.num_programs(0) - 1)
    def _():
        pooled = pool_ref[...] / jnp.maximum(cnt_ref[...], 1.0)
        a = jnp.maximum(
            jnp.dot(pooled, wm1_ref[...], preferred_element_type=jnp.float32)
            + bm1_ref[...], 0.0)
        out_ref[...] = (jnp.dot(a, wm2_ref[...],
                                preferred_element_type=jnp.float32)
                        + bm2_ref[...])


def _tcc(agg2, hs2, dinv, b2, batch_r, wm1, bm1, wm2p, bm2p):
    return pl.pallas_call(
        _tcc_body,
        grid=(NBLK,),
        in_specs=[
            pl.BlockSpec((NC, BLK, H), lambda i: (0, i, 0)),
            pl.BlockSpec((BLK, H), lambda i: (i, 0)),
            pl.BlockSpec((BLK, 1), lambda i: (i, 0)),
            pl.BlockSpec((1, H), lambda i: (0, 0)),
            pl.BlockSpec((1, 1, BLK), lambda i: (i, 0, 0)),
            pl.BlockSpec((H, H), lambda i: (0, 0)),
            pl.BlockSpec((1, H), lambda i: (0, 0)),
            pl.BlockSpec((H, H), lambda i: (0, 0)),
            pl.BlockSpec((1, H), lambda i: (0, 0)),
        ],
        out_specs=pl.BlockSpec((G, H), lambda i: (0, 0)),
        out_shape=jax.ShapeDtypeStruct((G, H), jnp.float32),
        scratch_shapes=[
            pltpu.VMEM((G, H), jnp.float32),
            pltpu.VMEM((G, H), jnp.float32),
        ],
    )(agg2, hs2, dinv, b2, batch_r, wm1, bm1, wm2p, bm2p)


def kernel(x, edge_index, batch, W1, b1, W2, b2, Wm1, bm1, Wm2, bm2):
    e = edge_index.shape[1]
    epw = -(-e // NW)
    nchunks = -(-epw // K)
    if nchunks % 2:
        nchunks += 1
    e_pad = NW * nchunks * K

    src = jnp.concatenate(
        [edge_index[0], jnp.full((e_pad - e,), N, dtype=jnp.int32)])
    dst = jnp.concatenate(
        [edge_index[1], jnp.full((e_pad - e,), N, dtype=jnp.int32)])
    src2 = src.reshape(NW * nchunks, K)
    dst2 = dst.reshape(NW * nchunks, K)
    sd3 = jnp.stack([src2, dst2], axis=1)

    per_pair = 2 * nchunks
    n0 = (per_pair * 11 // 16) // 2 * 2
    n1 = per_pair - n0

    x_pad = jnp.concatenate(
        [x, jnp.zeros((NPAD - N, D), dtype=jnp.float32)])
    batch_pad = jnp.concatenate(
        [batch, jnp.full((NPAD - N,), G, dtype=jnp.int32)]).reshape(NBLK, 1, BLK)

    ones128 = jnp.ones((K, 128), dtype=jnp.float32)
    zeros128 = jnp.zeros((NPAD, 128), dtype=jnp.float32)

    b1r = b1.reshape(1, H)
    b2r = b2.reshape(1, H)
    bm1r = bm1.reshape(1, H)
    wm2p = jnp.zeros((H, H), dtype=jnp.float32).at[:, :C].set(Wm2)
    bm2p = jnp.zeros((1, H), dtype=jnp.float32).at[0, :C].set(bm2)

    deg2 = _make_deg_kernel(nchunks)(dst2, ones128, zeros128)
    hs1, dinv = _tca(x_pad, deg2, W1)
    total_rows = NW * nchunks
    agg1 = _make_agg_kernel(n0, n1, total_rows)(hs1, sd3)
    hs2 = _tcb(agg1, hs1, dinv, b1r, W2)
    agg2 = _make_agg_kernel(n0, n1, total_rows)(hs2, sd3)
    outp = _tcc(agg2, hs2, dinv, b2r, batch_pad, Wm1, bm1r, wm2p, bm2p)
    return outp[:, :C]


# speedup vs baseline: 1.2663x; 1.0966x over previous
import functools

import jax
import jax.numpy as jnp
from jax import lax
from jax.experimental import pallas as pl
from jax.experimental.pallas import tpu as pltpu
from jax.experimental.pallas import tpu_sc as plsc

N = 10000
D = 128
H = 128
C = 2
G = 128

NC = 2
NS = 16
NW = NC * NS
K = 128
HK = 64
NPAD = 10240
RPT = NPAD // NS
BLK = 1024
NBLK = NPAD // BLK

_sc_mesh = plsc.VectorSubcoreMesh(core_axis_name="c", subcore_axis_name="s")
_sc_mesh1 = plsc.VectorSubcoreMesh(core_axis_name="c", subcore_axis_name="s",
                                   num_cores=1)


def _num_chunks(e_pad):
    return e_pad // (NW * K)


def _deg_body(nchunks, dst_hbm, ones_hbm, zeros_hbm, out_hbm,
              dst_v, ones_v, deg_sp, sem):
    c = lax.axis_index("c")
    s = lax.axis_index("s")
    w = s * NC + c
    pltpu.sync_copy(dst_hbm.at[pl.ds(w * nchunks, nchunks)], dst_v)
    pltpu.sync_copy(ones_hbm, ones_v)
    pltpu.sync_copy(zeros_hbm.at[pl.ds(s * RPT, RPT)],
                    deg_sp.at[pl.ds(s * RPT, RPT)])
    plsc.subcore_barrier()

    def body(j, carry):
        pltpu.sync_copy(ones_v, deg_sp.at[dst_v.at[j]], add=True)
        return carry

    lax.fori_loop(0, nchunks, body, 0)
    plsc.subcore_barrier()
    pltpu.sync_copy(deg_sp.at[pl.ds(s * RPT, RPT)],
                    out_hbm.at[c, pl.ds(s * RPT, RPT)])


def _make_deg_kernel(nchunks):
    return functools.partial(
        pl.kernel,
        out_type=jax.ShapeDtypeStruct((NC, NPAD, 128), jnp.float32),
        mesh=_sc_mesh,
        scratch_types=[
            pltpu.VMEM((nchunks, K), jnp.int32),
            pltpu.VMEM((K, 128), jnp.float32),
            pltpu.VMEM_SHARED((NPAD, 128), jnp.float32),
            pltpu.SemaphoreType.DMA,
        ],
    )(functools.partial(_deg_body, nchunks))


def _gather_halves(hs_hbm, sd_r, slot, buf, sem):
    buf_lo = buf.at[pl.ds(0, HK)]
    buf_hi = buf.at[pl.ds(HK, HK)]
    pltpu.async_copy(hs_hbm.at[sd_r.at[slot, 0, pl.ds(0, HK)]], buf_lo, sem)
    pltpu.async_copy(hs_hbm.at[sd_r.at[slot, 0, pl.ds(HK, HK)]], buf_hi, sem)


def _wait_halves(hs_hbm, sd_r, slot, buf, sem):
    buf_lo = buf.at[pl.ds(0, HK)]
    buf_hi = buf.at[pl.ds(HK, HK)]
    pltpu.make_async_copy(hs_hbm.at[sd_r.at[slot, 0, pl.ds(0, HK)]],
                          buf_lo, sem).wait()
    pltpu.make_async_copy(hs_hbm.at[sd_r.at[slot, 0, pl.ds(HK, HK)]],
                          buf_hi, sem).wait()


def _agg_body(n0, n1, total_rows, hs_hbm, sd_hbm, out_hbm,
              sd_r, buf_a, buf_b, agg_sp, gsem_a, gsem_b, isem):
    c = lax.axis_index("c")
    s = lax.axis_index("s")
    base = jnp.where(c == 0, s * n0, NS * n0 + s * n1)
    trips = jnp.where(c == 0, n0, n1)
    npairs = trips // 2

    def zbody(r, carry):
        for col in range(8):
            buf_a[r, pl.ds(col * 16, 16)] = jnp.zeros((16,), jnp.float32)
        return carry

    lax.fori_loop(0, K, zbody, 0)
    for blk in range(RPT // K):
        pltpu.sync_copy(buf_a, agg_sp.at[pl.ds(s * RPT + blk * K, K)])
    plsc.subcore_barrier()

    @pl.when(trips > 0)
    def _():
        pltpu.sync_copy(sd_hbm.at[pl.ds(base, 2)], sd_r.at[pl.ds(0, 2)])
        _gather_halves(hs_hbm, sd_r, 0, buf_a, gsem_a)
        _gather_halves(hs_hbm, sd_r, 1, buf_b, gsem_b)
        pltpu.async_copy(sd_hbm.at[pl.ds(base + 2, 2)], sd_r.at[pl.ds(2, 2)],
                         isem)

    def body(i, carry):
        j = base + 2 * i
        cur = (i % 2) * 2
        nxt = ((i + 1) % 2) * 2
        pltpu.make_async_copy(sd_hbm.at[pl.ds(base, 2)],
                              sd_r.at[pl.ds(0, 2)], isem).wait()

        _wait_halves(hs_hbm, sd_r, cur, buf_a, gsem_a)
        pltpu.sync_copy(buf_a, agg_sp.at[sd_r.at[cur, 1]], add=True)

        @pl.when(2 * i + 2 < trips)
        def _():
            _gather_halves(hs_hbm, sd_r, nxt, buf_a, gsem_a)

        _wait_halves(hs_hbm, sd_r, cur + 1, buf_b, gsem_b)
        pltpu.sync_copy(buf_b, agg_sp.at[sd_r.at[cur + 1, 1]], add=True)

        @pl.when(2 * i + 3 < trips)
        def _():
            _gather_halves(hs_hbm, sd_r, nxt + 1, buf_b, gsem_b)

        @pl.when(i + 1 < npairs)
        def _():
            row = jnp.minimum(j + 4, total_rows - 2)
            pltpu.async_copy(sd_hbm.at[pl.ds(row, 2)],
                             sd_r.at[pl.ds(cur, 2)], isem)
        return carry

    lax.fori_loop(0, npairs, body, 0)
    plsc.subcore_barrier()
    pltpu.sync_copy(agg_sp.at[pl.ds(s * RPT, RPT)],
                    out_hbm.at[c, pl.ds(s * RPT, RPT)])


def _make_agg_kernel(n0, n1, total_rows):
    return functools.partial(
        pl.kernel,
        out_type=jax.ShapeDtypeStruct((NC, NPAD, 128), jnp.float32),
        mesh=_sc_mesh,
        scratch_types=[
            pltpu.VMEM((4, 2, K), jnp.int32),
            pltpu.VMEM((K, 128), jnp.float32),
            pltpu.VMEM((K, 128), jnp.float32),
            pltpu.VMEM_SHARED((NPAD, 128), jnp.float32),
            pltpu.SemaphoreType.DMA,
            pltpu.SemaphoreType.DMA,
            pltpu.SemaphoreType.DMA,
        ],
    )(functools.partial(_agg_body, n0, n1, total_rows))


def _tca_body(x_ref, deg_ref, w1_ref, hs_ref, dinv_ref):
    deg = deg_ref[0, :, 0:1] + deg_ref[1, :, 0:1] + 1.0
    dinv = lax.rsqrt(deg)
    h = jnp.dot(x_ref[...], w1_ref[...], preferred_element_type=jnp.float32)
    hs_ref[...] = h * dinv
    dinv_ref[...] = dinv


def _tca(x_pad, deg2, w1):
    return pl.pallas_call(
        _tca_body,
        grid=(NBLK,),
        in_specs=[
            pl.BlockSpec((BLK, D), lambda i: (i, 0)),
            pl.BlockSpec((NC, BLK, 128), lambda i: (0, i, 0)),
            pl.BlockSpec((D, H), lambda i: (0, 0)),
        ],
        out_specs=[
            pl.BlockSpec((BLK, H), lambda i: (i, 0)),
            pl.BlockSpec((BLK, 1), lambda i: (i, 0)),
        ],
        out_shape=[
            jax.ShapeDtypeStruct((NPAD, H), jnp.float32),
            jax.ShapeDtypeStruct((NPAD, 1), jnp.float32),
        ],
    )(x_pad, deg2, w1)


def _tcb_body(agg_ref, hs1_ref, dinv_ref, b1_ref, w2_ref, hs2_ref):
    dinv = dinv_ref[...]
    tot = agg_ref[0] + agg_ref[1] + hs1_ref[...]
    h1r = jnp.maximum(dinv * tot + b1_ref[...], 0.0)
    hs2_ref[...] = dinv * jnp.dot(h1r, w2_ref[...],
                                  preferred_element_type=jnp.float32)


def _tcb(agg1, hs1, dinv, b1, w2):
    return pl.pallas_call(
        _tcb_body,
        grid=(NBLK,),
        in_specs=[
            pl.BlockSpec((NC, BLK, H), lambda i: (0, i, 0)),
            pl.BlockSpec((BLK, H), lambda i: (i, 0)),
            pl.BlockSpec((BLK, 1), lambda i: (i, 0)),
            pl.BlockSpec((1, H), lambda i: (0, 0)),
            pl.BlockSpec((H, H), lambda i: (0, 0)),
        ],
        out_specs=pl.BlockSpec((BLK, H), lambda i: (i, 0)),
        out_shape=jax.ShapeDtypeStruct((NPAD, H), jnp.float32),
    )(agg1, hs1, dinv, b1, w2)


def _tcc_body(agg_ref, hs2_ref, dinv_ref, b2_ref, batch_ref,
              wm1_ref, bm1_ref, wm2_ref, bm2_ref, out_ref,
              pool_ref, cnt_ref):
    i = pl.program_id(0)
    h2 = dinv_ref[...] * (agg_ref[0] + agg_ref[1] + hs2_ref[...]) + b2_ref[...]
    bb = batch_ref[0]
    gid = lax.broadcasted_iota(jnp.int32, (G, BLK), 0)
    eq = (gid == bb).astype(jnp.float32)
    ps = jnp.dot(eq, h2, preferred_element_type=jnp.float32)
    cs = jnp.sum(eq, axis=1, keepdims=True)

    @pl.when(i == 0)
    def _():
        pool_ref[...] = ps
        cnt_ref[...] = jnp.broadcast_to(cs, (G, H))

    @pl.when(i > 0)
    def _():
        pool_ref[...] += ps
        cnt_ref[...] += jnp.broadcast_to(cs, (G, H))

    @pl.when(i == pl.num_programs(0) - 1)
    def _():
        pooled = pool_ref[...] / jnp.maximum(cnt_ref[...], 1.0)
        a = jnp.maximum(
            jnp.dot(pooled, wm1_ref[...], preferred_element_type=jnp.float32)
            + bm1_ref[...], 0.0)
        out_ref[...] = (jnp.dot(a, wm2_ref[...],
                                preferred_element_type=jnp.float32)
                        + bm2_ref[...])


def _tcc(agg2, hs2, dinv, b2, batch_r, wm1, bm1, wm2p, bm2p):
    return pl.pallas_call(
        _tcc_body,
        grid=(NBLK,),
        in_specs=[
            pl.BlockSpec((NC, BLK, H), lambda i: (0, i, 0)),
            pl.BlockSpec((BLK, H), lambda i: (i, 0)),
            pl.BlockSpec((BLK, 1), lambda i: (i, 0)),
            pl.BlockSpec((1, H), lambda i: (0, 0)),
            pl.BlockSpec((1, 1, BLK), lambda i: (i, 0, 0)),
            pl.BlockSpec((H, H), lambda i: (0, 0)),
            pl.BlockSpec((1, H), lambda i: (0, 0)),
            pl.BlockSpec((H, H), lambda i: (0, 0)),
            pl.BlockSpec((1, H), lambda i: (0, 0)),
        ],
        out_specs=pl.BlockSpec((G, H), lambda i: (0, 0)),
        out_shape=jax.ShapeDtypeStruct((G, H), jnp.float32),
        scratch_shapes=[
            pltpu.VMEM((G, H), jnp.float32),
            pltpu.VMEM((G, H), jnp.float32),
        ],
    )(agg2, hs2, dinv, b2, batch_r, wm1, bm1, wm2p, bm2p)


def kernel(x, edge_index, batch, W1, b1, W2, b2, Wm1, bm1, Wm2, bm2):
    e = edge_index.shape[1]
    epw = -(-e // NW)
    nchunks = -(-epw // K)
    if nchunks % 2:
        nchunks += 1
    e_pad = NW * nchunks * K

    src = jnp.concatenate(
        [edge_index[0], jnp.full((e_pad - e,), N, dtype=jnp.int32)])
    dst = jnp.concatenate(
        [edge_index[1], jnp.full((e_pad - e,), N, dtype=jnp.int32)])
    src2 = src.reshape(NW * nchunks, K)
    dst2 = dst.reshape(NW * nchunks, K)
    sd3 = jnp.stack([src2, dst2], axis=1)

    per_pair = 2 * nchunks
    n0 = (per_pair * 7 // 8) // 2 * 2
    n1 = per_pair - n0

    x_pad = jnp.concatenate(
        [x, jnp.zeros((NPAD - N, D), dtype=jnp.float32)])
    batch_pad = jnp.concatenate(
        [batch, jnp.full((NPAD - N,), G, dtype=jnp.int32)]).reshape(NBLK, 1, BLK)

    ones128 = jnp.ones((K, 128), dtype=jnp.float32)
    zeros128 = jnp.zeros((NPAD, 128), dtype=jnp.float32)

    b1r = b1.reshape(1, H)
    b2r = b2.reshape(1, H)
    bm1r = bm1.reshape(1, H)
    wm2p = jnp.zeros((H, H), dtype=jnp.float32).at[:, :C].set(Wm2)
    bm2p = jnp.zeros((1, H), dtype=jnp.float32).at[0, :C].set(bm2)

    deg2 = _make_deg_kernel(nchunks)(dst2, ones128, zeros128)
    hs1, dinv = _tca(x_pad, deg2, W1)
    total_rows = NW * nchunks
    agg1 = _make_agg_kernel(n0, n1, total_rows)(hs1, sd3)
    hs2 = _tcb(agg1, hs1, dinv, b1r, W2)
    agg2 = _make_agg_kernel(n0, n1, total_rows)(hs2, sd3)
    outp = _tcc(agg2, hs2, dinv, b2r, batch_pad, Wm1, bm1r, wm2p, bm2p)
    return outp[:, :C]
